# Initial kernel scaffold; baseline (speedup 1.0000x reference)
import jax
import jax.numpy as jnp
from jax.experimental import pallas as pl


def kernel(det_boxes, det_scores, det_boxes_3d, det_scores_3d, embeddings, extrinsics, det_class_ids):
    raise NotImplementedError("write your pallas kernel here")


# speedup vs baseline: 18.1803x; 18.1803x over previous
import jax
import jax.numpy as jnp
from jax.experimental import pallas as pl
from jax.experimental.pallas import tpu as pltpu

_CLASS_RANGE = (40., 40., 40., 50., 50., 50., 50., 50., 50., 30., 30.)
_IOU_THR = 0.3
_B = 128
_NP = 5120
_NB = _NP // _B


def _prep_kernel(b3t_ref, s_ref, s3_ref, cls_ref, ext_ref,
                 scores_ref, feat_ref, validf_ref, b3o_ref):
    cx = b3t_ref[0:1, :]
    cy = b3t_ref[1:2, :]
    cz = b3t_ref[2:3, :]
    clsf = cls_ref[0:1, :].astype(jnp.float32)
    rng = jnp.zeros_like(cx)
    for k, r in enumerate(_CLASS_RANGE):
        rng = jnp.where(clsf == float(k), r, rng)
    dist = jnp.sqrt(cx * cx + cy * cy + cz * cz)
    validf = (dist < rng).astype(jnp.float32)
    validf_ref[0:1, :] = validf
    sc = s_ref[0:1, :] * s3_ref[0:1, :] * validf
    scores_ref[0:1, :] = sc

    def _bf(v):
        return v.astype(jnp.bfloat16).astype(jnp.float32)

    R = [[ext_ref[i, j] for j in range(3)] for i in range(3)]
    t = [ext_ref[i, 3] for i in range(3)]
    cxb, cyb, czb = _bf(cx), _bf(cy), _bf(cz)
    cw = [cxb * R[i][0] + cyb * R[i][1] + czb * R[i][2] + t[i]
          for i in range(3)]
    for i in range(3):
        b3o_ref[i:i + 1, :] = cw[i]
        b3o_ref[3 + i:4 + i, :] = b3t_ref[3 + i:4 + i, :]
    o6 = _bf(b3t_ref[6:7, :])
    o7 = _bf(b3t_ref[7:8, :])
    o8 = _bf(b3t_ref[8:9, :])
    v9 = _bf(b3t_ref[9:10, :])
    v10 = _bf(b3t_ref[10:11, :])
    v11 = _bf(b3t_ref[11:12, :])
    for i in range(3):
        b3o_ref[6 + i:7 + i, :] = o6 * R[i][0] + o7 * R[i][1] + o8 * R[i][2]
        b3o_ref[9 + i:10 + i, :] = v9 * R[i][0] + v10 * R[i][1] + v11 * R[i][2]

    w = jnp.abs(b3t_ref[3:4, :]) + 0.5
    l = jnp.abs(b3t_ref[5:6, :]) + 0.5
    x = cw[0]
    z = cw[2]
    feat_ref[0:1, :] = x - w * 0.5
    feat_ref[1:2, :] = x + w * 0.5
    feat_ref[2:3, :] = z - l * 0.5
    feat_ref[3:4, :] = z + l * 0.5
    feat_ref[4:5, :] = w * l


def _nms_kernel(featr_ref, featc_ref, clsr_ref, clsc_ref, keep_ref, m_buf):
    b = pl.program_id(0)

    @pl.when(b == 0)
    def _init():
        keep_ref[:, :] = jnp.zeros((_NP, 1), jnp.float32)

    blk = pl.ds(b * _B, _B)
    bx1 = featr_ref[0:1, blk]
    bx2 = featr_ref[1:2, blk]
    bz1 = featr_ref[2:3, blk]
    bz2 = featr_ref[3:4, blk]
    bar = featr_ref[4:5, blk]
    bcls = clsr_ref[0:1, blk]

    def _sup_chunk(ch):
        ax1 = featc_ref[ch, 0:1]
        ax2 = featc_ref[ch, 1:2]
        az1 = featc_ref[ch, 2:3]
        az2 = featc_ref[ch, 3:4]
        aar = featc_ref[ch, 4:5]
        acls = clsc_ref[ch, 0:1]
        ix = jnp.maximum(jnp.minimum(ax2, bx2) - jnp.maximum(ax1, bx1), 0.0)
        iz = jnp.maximum(jnp.minimum(az2, bz2) - jnp.maximum(az1, bz1), 0.0)
        inter = ix * iz
        union = aar + bar - inter
        iou = inter / jnp.maximum(union, 1e-9)
        return jnp.logical_and(iou > _IOU_THR, acls == bcls).astype(jnp.float32)

    m_buf[:, :] = _sup_chunk(blk)

    def chunk_body(kb, pre):
        ch = pl.ds(kb * _B, _B)
        sup = _sup_chunk(ch)
        kkeep = keep_ref[ch, 0:1]
        return jnp.maximum(pre, jnp.max(sup * kkeep, axis=0, keepdims=True))

    pre = jax.lax.fori_loop(0, _NB, chunk_body,
                            jnp.zeros((1, _B), jnp.float32))
    lane = jax.lax.broadcasted_iota(jnp.int32, (1, _B), 1)

    def inner(i, acc):
        dead = jnp.sum(jnp.where(lane == i, acc, 0.0))
        alive = (dead == 0.0).astype(jnp.float32)
        keep_ref[pl.ds(b * _B + i, 1), 0:1] = alive.reshape(1, 1)
        row = m_buf[pl.ds(i, 1), :]
        return jnp.maximum(acc, alive * row)

    jax.lax.fori_loop(0, _B, inner, pre)


def _out_kernel(boxes_ref, b3_ref, sc_ref, emb_ref, keep_ref, out_ref):
    k = keep_ref[:, 0:1]
    out_ref[:, 0:4] = boxes_ref[:, :] * k
    out_ref[:, 4:16] = b3_ref[:, :] * k
    out_ref[:, 16:17] = sc_ref[:, :] * k
    out_ref[:, 17:273] = emb_ref[:, :] * k


def _run_prep(b3t, s, s3, cls, extrinsics):
    n = s.shape[1]
    rot = extrinsics[:3, :3].astype(jnp.bfloat16).astype(jnp.float32)
    extrinsics = jnp.concatenate(
        [jnp.concatenate([rot, extrinsics[:3, 3:4]], axis=1),
         extrinsics[3:4, :]], axis=0)
    return pl.pallas_call(
        _prep_kernel,
        in_specs=[
            pl.BlockSpec(memory_space=pltpu.VMEM),
            pl.BlockSpec(memory_space=pltpu.VMEM),
            pl.BlockSpec(memory_space=pltpu.VMEM),
            pl.BlockSpec(memory_space=pltpu.VMEM),
            pl.BlockSpec(memory_space=pltpu.SMEM),
        ],
        out_shape=(
            jax.ShapeDtypeStruct((1, n), jnp.float32),
            jax.ShapeDtypeStruct((5, n), jnp.float32),
            jax.ShapeDtypeStruct((1, n), jnp.float32),
            jax.ShapeDtypeStruct((12, n), jnp.float32),
        ),
    )(b3t, s, s3, cls, extrinsics)


def _run_nms(feat_s, cls_s):
    featc = feat_s.T
    clsc = cls_s.T
    return pl.pallas_call(
        _nms_kernel,
        grid=(_NB,),
        in_specs=[
            pl.BlockSpec(feat_s.shape, lambda b: (0, 0)),
            pl.BlockSpec(featc.shape, lambda b: (0, 0)),
            pl.BlockSpec(cls_s.shape, lambda b: (0, 0)),
            pl.BlockSpec(clsc.shape, lambda b: (0, 0)),
        ],
        out_specs=pl.BlockSpec((_NP, 1), lambda b: (0, 0)),
        out_shape=jax.ShapeDtypeStruct((_NP, 1), jnp.float32),
        scratch_shapes=[pltpu.VMEM((_B, _B), jnp.float32)],
        compiler_params=pltpu.CompilerParams(
            dimension_semantics=("arbitrary",)),
    )(feat_s, featc, cls_s, clsc)


def kernel(det_boxes, det_scores, det_boxes_3d, det_scores_3d, embeddings,
           extrinsics, det_class_ids):
    n = det_scores.shape[0]
    b3t = det_boxes_3d.T
    s = det_scores.reshape(1, n)
    s3 = det_scores_3d.reshape(1, n)
    cls = det_class_ids.astype(jnp.int32).reshape(1, n)

    scores, feat, validf, b3o = _run_prep(b3t, s, s3, cls, extrinsics)

    scores1 = scores[0]
    order = jnp.argsort(-scores1)
    pad = _NP - n
    feat_s = jnp.pad(feat[:, order], ((0, 0), (0, pad)))
    cls_s = jnp.pad(cls[0, order].astype(jnp.float32), (0, pad),
                    constant_values=-1.0).reshape(1, _NP)
    keep = _run_nms(feat_s, cls_s)

    keep_orig = jnp.zeros((n,), jnp.float32).at[order].set(keep[:n, 0])
    keepf = (keep_orig * validf[0]).reshape(n, 1)

    out = pl.pallas_call(
        _out_kernel,
        out_shape=jax.ShapeDtypeStruct((n, 273), jnp.float32),
    )(det_boxes, b3o.T, scores1.reshape(n, 1), embeddings, keepf)
    return out
